# baseline (device time: 46804 ns/iter reference)
import jax
import jax.numpy as jnp
from jax import lax
from jax.experimental import pallas as pl
from jax.experimental.pallas import tpu as pltpu

N_DEV = 16
BLK = 256

_SEND_ORDER = [12, 13, 14, 15, 8, 9, 10, 11, 4, 5, 6, 7, 1, 2, 3]


def kernel(x, w_mat):
    k_total, k_shard = x.shape
    _, n = w_mat.shape
    m_per = k_total // N_DEV

    def body(x_ref, w_ref, out_ref, xsend_ref, xrecv_ref, wbuf_ref,
             send_sems, recv_sems, wcopy_sems):
        my_i = lax.axis_index("i")

        for si, d in enumerate(_SEND_ORDER):
            dst = lax.rem(my_i + d, N_DEV)
            xsend_ref[pl.ds(dst * m_per, m_per), :] = (
                x_ref[pl.ds(dst * m_per, m_per), :].astype(jnp.bfloat16)
            )
            pltpu.make_async_remote_copy(
                src_ref=xsend_ref.at[pl.ds(dst * m_per, m_per), :],
                dst_ref=xrecv_ref.at[my_i],
                send_sem=send_sems.at[si],
                recv_sem=recv_sems.at[my_i],
                device_id=(dst,),
                device_id_type=pl.DeviceIdType.MESH,
            ).start()

        def w_src(t):
            return lax.rem(my_i + (N_DEV - t), N_DEV) if t else my_i

        W_SLOTS = 4

        def start_wcopy(t):
            pltpu.make_async_copy(
                w_ref.at[pl.ds(w_src(t) * BLK, BLK), :],
                wbuf_ref.at[t % W_SLOTS],
                wcopy_sems.at[t % W_SLOTS],
            ).start()

        def wait_wcopy(t):
            pltpu.make_async_copy(
                w_ref.at[pl.ds(w_src(t) * BLK, BLK), :],
                wbuf_ref.at[t % W_SLOTS],
                wcopy_sems.at[t % W_SLOTS],
            ).wait()

        for t in range(W_SLOTS - 1):
            start_wcopy(t)

        for t in range(N_DEV):
            wait_wcopy(t)
            if t + W_SLOTS - 1 < N_DEV:
                start_wcopy(t + W_SLOTS - 1)
            if t == 0:
                xblk = x_ref[pl.ds(my_i * m_per, m_per), :]
                out_ref[:, :] = jnp.dot(
                    xblk, wbuf_ref[0],
                    preferred_element_type=jnp.float32,
                )
            else:
                src = lax.rem(my_i + N_DEV - t, N_DEV)
                pltpu.make_async_remote_copy(
                    src_ref=xsend_ref.at[pl.ds(0, m_per), :],
                    dst_ref=xrecv_ref.at[src],
                    send_sem=send_sems.at[0],
                    recv_sem=recv_sems.at[src],
                    device_id=(src,),
                    device_id_type=pl.DeviceIdType.MESH,
                ).wait_recv()
                out_ref[:, :] += jnp.dot(
                    xrecv_ref[src].astype(jnp.float32), wbuf_ref[t % W_SLOTS],
                    preferred_element_type=jnp.float32,
                )

        out_ref[:, :] = jnp.maximum(out_ref[:, :], 0.0)

        for si, d in enumerate(_SEND_ORDER):
            dst = lax.rem(my_i + d, N_DEV)
            pltpu.make_async_remote_copy(
                src_ref=xsend_ref.at[pl.ds(dst * m_per, m_per), :],
                dst_ref=xrecv_ref.at[my_i],
                send_sem=send_sems.at[si],
                recv_sem=recv_sems.at[my_i],
                device_id=(dst,),
                device_id_type=pl.DeviceIdType.MESH,
            ).wait_send()

    return pl.pallas_call(
        body,
        out_shape=jax.ShapeDtypeStruct((m_per, n), jnp.float32),
        in_specs=[
            pl.BlockSpec(memory_space=pltpu.VMEM),
            pl.BlockSpec(memory_space=pltpu.MemorySpace.HBM),
        ],
        out_specs=pl.BlockSpec(memory_space=pltpu.VMEM),
        scratch_shapes=[
            pltpu.VMEM((k_total, k_shard), jnp.bfloat16),
            pltpu.VMEM((N_DEV, m_per, BLK), jnp.bfloat16),
            pltpu.VMEM((4, BLK, n), jnp.float32),
            pltpu.SemaphoreType.DMA((N_DEV - 1,)),
            pltpu.SemaphoreType.DMA((N_DEV,)),
            pltpu.SemaphoreType.DMA((4,)),
        ],
        compiler_params=pltpu.CompilerParams(
            vmem_limit_bytes=100 * 1024 * 1024,
        ),
    )(x, w_mat)


# device time: 45744 ns/iter; 1.0232x vs baseline; 1.0232x over previous
import jax
import jax.numpy as jnp
from jax import lax
from jax.experimental import pallas as pl
from jax.experimental.pallas import tpu as pltpu

N_DEV = 16
BLK = 256

_SEND_ORDER = [12, 13, 14, 15, 8, 9, 10, 11, 4, 5, 6, 7, 1, 2, 3]


def kernel(x, w_mat):
    k_total, k_shard = x.shape
    _, n = w_mat.shape
    m_per = k_total // N_DEV

    def body(x_ref, w_ref, out_ref, xsend_ref, xrecv_ref, wbuf_ref,
             send_sems, recv_sems, wcopy_sems):
        my_i = lax.axis_index("i")

        xsend_ref[:, :] = x_ref[:, :].astype(jnp.bfloat16)

        for si, d in enumerate(_SEND_ORDER):
            dst = lax.rem(my_i + d, N_DEV)
            pltpu.make_async_remote_copy(
                src_ref=xsend_ref.at[pl.ds(dst * m_per, m_per), :],
                dst_ref=xrecv_ref.at[my_i],
                send_sem=send_sems.at[si],
                recv_sem=recv_sems.at[my_i],
                device_id=(dst,),
                device_id_type=pl.DeviceIdType.MESH,
            ).start()

        def w_src(t):
            return lax.rem(my_i + (N_DEV - t), N_DEV) if t else my_i

        W_SLOTS = 6

        def start_wcopy(t):
            pltpu.make_async_copy(
                w_ref.at[pl.ds(w_src(t) * BLK, BLK), :],
                wbuf_ref.at[t % W_SLOTS],
                wcopy_sems.at[t % W_SLOTS],
            ).start()

        def wait_wcopy(t):
            pltpu.make_async_copy(
                w_ref.at[pl.ds(w_src(t) * BLK, BLK), :],
                wbuf_ref.at[t % W_SLOTS],
                wcopy_sems.at[t % W_SLOTS],
            ).wait()

        for t in range(W_SLOTS - 1):
            start_wcopy(t)

        for t in range(N_DEV):
            wait_wcopy(t)
            if t + W_SLOTS - 1 < N_DEV:
                start_wcopy(t + W_SLOTS - 1)
            if t == 0:
                xblk = x_ref[pl.ds(my_i * m_per, m_per), :]
                out_ref[:, :] = jnp.dot(
                    xblk, wbuf_ref[0],
                    preferred_element_type=jnp.float32,
                )
            else:
                src = lax.rem(my_i + N_DEV - t, N_DEV)
                pltpu.make_async_remote_copy(
                    src_ref=xsend_ref.at[pl.ds(0, m_per), :],
                    dst_ref=xrecv_ref.at[src],
                    send_sem=send_sems.at[0],
                    recv_sem=recv_sems.at[src],
                    device_id=(src,),
                    device_id_type=pl.DeviceIdType.MESH,
                ).wait_recv()
                acc = out_ref[:, :] + jnp.dot(
                    xrecv_ref[src].astype(jnp.float32), wbuf_ref[t % W_SLOTS],
                    preferred_element_type=jnp.float32,
                )
                if t == N_DEV - 1:
                    acc = jnp.maximum(acc, 0.0)
                out_ref[:, :] = acc

        for si, d in enumerate(_SEND_ORDER):
            dst = lax.rem(my_i + d, N_DEV)
            pltpu.make_async_remote_copy(
                src_ref=xsend_ref.at[pl.ds(dst * m_per, m_per), :],
                dst_ref=xrecv_ref.at[my_i],
                send_sem=send_sems.at[si],
                recv_sem=recv_sems.at[my_i],
                device_id=(dst,),
                device_id_type=pl.DeviceIdType.MESH,
            ).wait_send()

    return pl.pallas_call(
        body,
        out_shape=jax.ShapeDtypeStruct((m_per, n), jnp.float32),
        in_specs=[
            pl.BlockSpec(memory_space=pltpu.VMEM),
            pl.BlockSpec(memory_space=pltpu.MemorySpace.HBM),
        ],
        out_specs=pl.BlockSpec(memory_space=pltpu.VMEM),
        scratch_shapes=[
            pltpu.VMEM((k_total, k_shard), jnp.bfloat16),
            pltpu.VMEM((N_DEV, m_per, BLK), jnp.bfloat16),
            pltpu.VMEM((6, BLK, n), jnp.float32),
            pltpu.SemaphoreType.DMA((N_DEV - 1,)),
            pltpu.SemaphoreType.DMA((N_DEV,)),
            pltpu.SemaphoreType.DMA((6,)),
        ],
        compiler_params=pltpu.CompilerParams(
            vmem_limit_bytes=100 * 1024 * 1024,
        ),
    )(x, w_mat)


# device time: 45380 ns/iter; 1.0314x vs baseline; 1.0080x over previous
import jax
import jax.numpy as jnp
from jax import lax
from jax.experimental import pallas as pl
from jax.experimental.pallas import tpu as pltpu

N_DEV = 16
BLK = 256

_SEND_ORDER = [12, 13, 14, 15, 8, 9, 10, 11, 4, 5, 6, 7, 1, 2, 3]


def kernel(x, w_mat):
    k_total, k_shard = x.shape
    _, n = w_mat.shape
    m_per = k_total // N_DEV

    def body(x_ref, w_ref, out_ref, xsend_ref, xrecv_ref, wbuf_ref, acc_ref,
             send_sems, recv_sems, wcopy_sems, ocopy_sem):
        my_i = lax.axis_index("i")

        xsend_ref[:, :] = x_ref[:, :].astype(jnp.bfloat16)

        for si, d in enumerate(_SEND_ORDER):
            dst = lax.rem(my_i + d, N_DEV)
            pltpu.make_async_remote_copy(
                src_ref=xsend_ref.at[pl.ds(dst * m_per, m_per), :],
                dst_ref=xrecv_ref.at[my_i],
                send_sem=send_sems.at[si],
                recv_sem=recv_sems.at[my_i],
                device_id=(dst,),
                device_id_type=pl.DeviceIdType.MESH,
            ).start()

        def w_src(t):
            return lax.rem(my_i + (N_DEV - t), N_DEV) if t else my_i

        W_SLOTS = 6

        def start_wcopy(t):
            pltpu.make_async_copy(
                w_ref.at[pl.ds(w_src(t) * BLK, BLK), :],
                wbuf_ref.at[t % W_SLOTS],
                wcopy_sems.at[t % W_SLOTS],
            ).start()

        def wait_wcopy(t):
            pltpu.make_async_copy(
                w_ref.at[pl.ds(w_src(t) * BLK, BLK), :],
                wbuf_ref.at[t % W_SLOTS],
                wcopy_sems.at[t % W_SLOTS],
            ).wait()

        for t in range(W_SLOTS - 1):
            start_wcopy(t)

        for t in range(N_DEV):
            wait_wcopy(t)
            if t + W_SLOTS - 1 < N_DEV:
                start_wcopy(t + W_SLOTS - 1)
            if t == 0:
                xblk = x_ref[pl.ds(my_i * m_per, m_per), :]
                acc_ref[:, :] = jnp.dot(
                    xblk, wbuf_ref[0],
                    preferred_element_type=jnp.float32,
                )
            else:
                src = lax.rem(my_i + N_DEV - t, N_DEV)
                pltpu.make_async_remote_copy(
                    src_ref=xsend_ref.at[pl.ds(0, m_per), :],
                    dst_ref=xrecv_ref.at[src],
                    send_sem=send_sems.at[0],
                    recv_sem=recv_sems.at[src],
                    device_id=(src,),
                    device_id_type=pl.DeviceIdType.MESH,
                ).wait_recv()
                acc = acc_ref[:, :] + jnp.dot(
                    xrecv_ref[src].astype(jnp.float32), wbuf_ref[t % W_SLOTS],
                    preferred_element_type=jnp.float32,
                )
                if t == N_DEV - 1:
                    acc = jnp.maximum(acc, 0.0)
                acc_ref[:, :] = acc

        ocopy = pltpu.make_async_copy(acc_ref, out_ref, ocopy_sem)
        ocopy.start()
        ocopy.wait()

        for si, d in enumerate(_SEND_ORDER):
            dst = lax.rem(my_i + d, N_DEV)
            pltpu.make_async_remote_copy(
                src_ref=xsend_ref.at[pl.ds(dst * m_per, m_per), :],
                dst_ref=xrecv_ref.at[my_i],
                send_sem=send_sems.at[si],
                recv_sem=recv_sems.at[my_i],
                device_id=(dst,),
                device_id_type=pl.DeviceIdType.MESH,
            ).wait_send()

    return pl.pallas_call(
        body,
        out_shape=jax.ShapeDtypeStruct((m_per, n), jnp.float32),
        in_specs=[
            pl.BlockSpec(memory_space=pltpu.VMEM),
            pl.BlockSpec(memory_space=pltpu.MemorySpace.HBM),
        ],
        out_specs=pl.BlockSpec(memory_space=pltpu.MemorySpace.HBM),
        scratch_shapes=[
            pltpu.VMEM((k_total, k_shard), jnp.bfloat16),
            pltpu.VMEM((N_DEV, m_per, BLK), jnp.bfloat16),
            pltpu.VMEM((6, BLK, n), jnp.float32),
            pltpu.VMEM((m_per, n), jnp.float32),
            pltpu.SemaphoreType.DMA((N_DEV - 1,)),
            pltpu.SemaphoreType.DMA((N_DEV,)),
            pltpu.SemaphoreType.DMA((6,)),
            pltpu.SemaphoreType.DMA,
        ],
        compiler_params=pltpu.CompilerParams(
            vmem_limit_bytes=100 * 1024 * 1024,
        ),
    )(x, w_mat)
